# baseline (speedup 1.0000x reference)
import functools

import jax
import jax.numpy as jnp
from jax import lax
from jax.experimental import pallas as pl
from jax.experimental.pallas import tpu as pltpu
from jax.experimental.pallas import tpu_sc as plsc

B = 8
C = 96
W = 224
R = B * C * W
E = C * W * W
NC, NS, L = 2, 16, 16
NW = NC * NS
WPS = NW // B
RPW = R // NW
KV = W // L

RB1 = 224
RPS = C * W
S_SC = 8960
RPW1 = S_SC // WPS
NJ1 = RPW1 // RB1 // 2
RBT = 448
NBT = (RPS - S_SC) // RBT
TBLK = RPS // RBT
RB2 = 96
NJ2 = RPW // RB2 // 2

_mesh = plsc.VectorSubcoreMesh(
    core_axis_name="c", subcore_axis_name="s", num_cores=NC, num_subcores=NS
)


def _worker_id():
    return lax.axis_index("s") * NC + lax.axis_index("c")


@functools.partial(
    pl.kernel,
    out_type=jax.ShapeDtypeStruct((NW, 3, L), jnp.float32),
    mesh=_mesh,
    scratch_types=[
        pltpu.VMEM((RB1, W), jnp.float32),
        pltpu.VMEM((RB1, W), jnp.float32),
        pltpu.VMEM((3, L), jnp.float32),
        pltpu.SemaphoreType.DMA,
        pltpu.SemaphoreType.DMA,
    ],
)
def _stats_kernel(x_hbm, part_hbm, buf_a, buf_b, part_v, sem_a, sem_b):
    wid = _worker_id()
    b = wid // WPS
    base = b * RPS + (wid % WPS) * RPW1

    def load(ci, buf, sem):
        return pltpu.make_async_copy(
            x_hbm.at[pl.ds(base + ci * RB1, RB1)], buf, sem
        )

    def chunk_stats(buf, acc):
        def row_body(j, c):
            neg, s1, s2 = c
            for k in range(KV):
                v = buf[j, pl.ds(k * L, L)]
                m = jnp.maximum(v, 0.0)
                neg = neg + lax.shift_right_logical(
                    lax.bitcast_convert_type(v, jnp.int32), 31
                )
                s1 = s1 + m
                s2 = s2 + m * m
            return (neg, s1, s2)

        return plsc.parallel_loop(0, RB1, 1, carry=acc)(row_body)

    load(0, buf_a, sem_a).start()

    def pair_body(j, acc):
        load(2 * j + 1, buf_b, sem_b).start()
        load(2 * j, buf_a, sem_a).wait()
        acc = chunk_stats(buf_a, acc)

        @pl.when(j < NJ1 - 1)
        def _():
            load(2 * j + 2, buf_a, sem_a).start()

        load(2 * j + 1, buf_b, sem_b).wait()
        return chunk_stats(buf_b, acc)

    zf = jnp.zeros((L,), jnp.float32)
    zi = jnp.zeros((L,), jnp.int32)
    neg, s1, s2 = lax.fori_loop(0, NJ1, pair_body, (zi, zf, zf))
    part_v[0, :] = neg.astype(jnp.float32)
    part_v[1, :] = s1
    part_v[2, :] = s2
    pltpu.sync_copy(part_v, part_hbm.at[wid])


def _tc_stats_body(x_ref, out_ref):
    bb = pl.program_id(0)
    i = pl.program_id(1)
    v = x_ref[...]
    valid = (v >= 0.0).astype(jnp.float32)
    m = jnp.maximum(v, 0.0)
    n = jnp.sum(valid)
    s1 = jnp.sum(m)
    s2 = jnp.sum(m * m)
    io_r = lax.broadcasted_iota(jnp.int32, (B, 128), 0)
    io_c = lax.broadcasted_iota(jnp.int32, (B, 128), 1)
    val = jnp.where(
        io_c == 0, n, jnp.where(io_c == 1, s1, jnp.where(io_c == 2, s2, 0.0))
    )
    vec = jnp.where(io_r == bb, val, 0.0)

    @pl.when((bb == 0) & (i == 0))
    def _():
        out_ref[...] = jnp.zeros((B, 128), jnp.float32)

    out_ref[...] = out_ref[...] + vec


_tc_stats = pl.pallas_call(
    _tc_stats_body,
    out_shape=jax.ShapeDtypeStruct((B, 128), jnp.float32),
    grid=(B, NBT),
    in_specs=[
        pl.BlockSpec((RBT, W), lambda b, i: (b * TBLK + S_SC // RBT + i, 0))
    ],
    out_specs=pl.BlockSpec((B, 128), lambda b, i: (0, 0)),
)


@functools.partial(
    pl.kernel,
    out_type=jax.ShapeDtypeStruct((R, W), jnp.float32),
    mesh=_mesh,
    scratch_types=[
        pltpu.VMEM((RB2, W), jnp.float32),
        pltpu.VMEM((RB2, W), jnp.float32),
        pltpu.VMEM((RB2, W), jnp.float32),
        pltpu.VMEM((RB2, W), jnp.float32),
        pltpu.VMEM((NW, 3, L), jnp.float32),
        pltpu.VMEM((B, 128), jnp.float32),
        pltpu.SemaphoreType.DMA,
        pltpu.SemaphoreType.DMA,
        pltpu.SemaphoreType.DMA,
        pltpu.SemaphoreType.DMA,
    ],
)
def _norm_kernel(
    x_hbm, part_hbm, ptc_hbm, out_hbm,
    in_a, in_b, out_a, out_b, part_v, ptc_v,
    lsem_a, lsem_b, ssem_a, ssem_b,
):
    wid = _worker_id()
    b = wid // WPS
    base = wid * RPW

    pltpu.sync_copy(part_hbm, part_v)
    pltpu.sync_copy(ptc_hbm, ptc_v)
    negv = jnp.zeros((L,), jnp.float32)
    s1v = jnp.zeros((L,), jnp.float32)
    s2v = jnp.zeros((L,), jnp.float32)
    for k in range(WPS):
        w = b * WPS + k
        negv = negv + part_v[w, 0, :]
        s1v = s1v + part_v[w, 1, :]
        s2v = s2v + part_v[w, 2, :]
    neg = negv[0]
    s1 = s1v[0]
    s2 = s2v[0]
    for j in range(1, L):
        neg = neg + negv[j]
        s1 = s1 + s1v[j]
        s2 = s2 + s2v[j]

    tcv = ptc_v[b, pl.ds(0, L)]
    n_sc = float(S_SC * W) - neg
    n_v = jnp.full((L,), n_sc, jnp.float32) + jnp.full((L,), tcv[0], jnp.float32)
    s1_v = jnp.full((L,), s1, jnp.float32) + jnp.full((L,), tcv[1], jnp.float32)
    s2_v = jnp.full((L,), s2, jnp.float32) + jnp.full((L,), tcv[2], jnp.float32)
    mean_v = s1_v / n_v
    var_v = (s2_v - s1_v * mean_v) / (n_v - 1.0)
    var_v = jnp.maximum(var_v, 1e-20)
    bits = lax.bitcast_convert_type(var_v, jnp.int32)
    r = lax.bitcast_convert_type(0x5F3759DF - (bits >> 1), jnp.float32)
    for _ in range(3):
        r = r * (1.5 - 0.5 * var_v * r * r)
    inv_v = 1.0 / (var_v * r + 1e-5)
    c_v = -mean_v * inv_v

    def load(ci, buf, sem):
        return pltpu.make_async_copy(
            x_hbm.at[pl.ds(base + ci * RB2, RB2)], buf, sem
        )

    def store(ci, buf, sem):
        return pltpu.make_async_copy(
            buf, out_hbm.at[pl.ds(base + ci * RB2, RB2)], sem
        )

    def chunk_norm(ibuf, obuf):
        def row_body(j):
            for k in range(KV):
                v = ibuf[j, pl.ds(k * L, L)]
                obuf[j, pl.ds(k * L, L)] = jnp.where(
                    v >= 0.0, v * inv_v + c_v, v
                )

        plsc.parallel_loop(0, RB2, 1)(row_body)

    load(0, in_a, lsem_a).start()

    def pair_body(j, carry):
        load(2 * j + 1, in_b, lsem_b).start()
        load(2 * j, in_a, lsem_a).wait()

        @pl.when(j > 0)
        def _():
            store(2 * j - 2, out_a, ssem_a).wait()

        chunk_norm(in_a, out_a)
        store(2 * j, out_a, ssem_a).start()

        @pl.when(j < NJ2 - 1)
        def _():
            load(2 * j + 2, in_a, lsem_a).start()

        load(2 * j + 1, in_b, lsem_b).wait()

        @pl.when(j > 0)
        def _():
            store(2 * j - 1, out_b, ssem_b).wait()

        chunk_norm(in_b, out_b)
        store(2 * j + 1, out_b, ssem_b).start()
        return carry

    lax.fori_loop(0, NJ2, pair_body, 0)
    store(2 * NJ2 - 2, out_a, ssem_a).wait()
    store(2 * NJ2 - 1, out_b, ssem_b).wait()


def kernel(x):
    x2 = x.reshape(R, W)
    part_sc = _stats_kernel(x2)
    part_tc = _tc_stats(x2)
    out = _norm_kernel(x2, part_sc, part_tc)
    return out.reshape(x.shape)


# speedup vs baseline: 1.2354x; 1.2354x over previous
import functools

import jax
import jax.numpy as jnp
from jax import lax
from jax.experimental import pallas as pl
from jax.experimental.pallas import tpu as pltpu
from jax.experimental.pallas import tpu_sc as plsc

B = 8
C = 96
W = 224
R = B * C * W
E = C * W * W
NC, NS, L = 2, 16, 16
NW = NC * NS
WPS = NW // B
RPW = R // NW
KV = W // L

RB1 = 224
RPS = C * W
S_SC = 8960
RPW1 = S_SC // WPS
NJ1 = RPW1 // RB1 // 2
RBT = 896
NBT = (RPS - S_SC) // RBT
TBLK = RPS // RBT
RB2 = 96
NJ2 = RPW // RB2 // 2

_mesh = plsc.VectorSubcoreMesh(
    core_axis_name="c", subcore_axis_name="s", num_cores=NC, num_subcores=NS
)


def _worker_id():
    return lax.axis_index("s") * NC + lax.axis_index("c")


@functools.partial(
    pl.kernel,
    out_type=jax.ShapeDtypeStruct((NW, 3, L), jnp.float32),
    mesh=_mesh,
    scratch_types=[
        pltpu.VMEM((RB1, W), jnp.float32),
        pltpu.VMEM((RB1, W), jnp.float32),
        pltpu.VMEM((3, L), jnp.float32),
        pltpu.SemaphoreType.DMA,
        pltpu.SemaphoreType.DMA,
    ],
)
def _stats_kernel(x_hbm, part_hbm, buf_a, buf_b, part_v, sem_a, sem_b):
    wid = _worker_id()
    b = wid // WPS
    base = b * RPS + (wid % WPS) * RPW1

    def load(ci, buf, sem):
        return pltpu.make_async_copy(
            x_hbm.at[pl.ds(base + ci * RB1, RB1)], buf, sem
        )

    def chunk_stats(buf, acc):
        def row_body(j, c):
            neg, s1, s2 = c
            for k in range(KV):
                v = buf[j, pl.ds(k * L, L)]
                m = jnp.maximum(v, 0.0)
                neg = neg + lax.shift_right_logical(
                    lax.bitcast_convert_type(v, jnp.int32), 31
                )
                s1 = s1 + m
                s2 = s2 + m * m
            return (neg, s1, s2)

        return plsc.parallel_loop(0, RB1, 1, carry=acc)(row_body)

    load(0, buf_a, sem_a).start()

    def pair_body(j, acc):
        load(2 * j + 1, buf_b, sem_b).start()
        load(2 * j, buf_a, sem_a).wait()
        acc = chunk_stats(buf_a, acc)

        @pl.when(j < NJ1 - 1)
        def _():
            load(2 * j + 2, buf_a, sem_a).start()

        load(2 * j + 1, buf_b, sem_b).wait()
        return chunk_stats(buf_b, acc)

    zf = jnp.zeros((L,), jnp.float32)
    zi = jnp.zeros((L,), jnp.int32)
    neg, s1, s2 = lax.fori_loop(0, NJ1, pair_body, (zi, zf, zf))
    part_v[0, :] = neg.astype(jnp.float32)
    part_v[1, :] = s1
    part_v[2, :] = s2
    pltpu.sync_copy(part_v, part_hbm.at[wid])


def _tc_stats_body(x_ref, out_ref):
    bb = pl.program_id(0)
    i = pl.program_id(1)
    v = x_ref[...]
    valid = (v >= 0.0).astype(jnp.float32)
    m = jnp.maximum(v, 0.0)
    n = jnp.sum(valid)
    s1 = jnp.sum(m)
    s2 = jnp.sum(m * m)

    @pl.when(i == 0)
    def _():
        for j in range(L):
            out_ref[bb, j] = 0.0

    out_ref[bb, 0] = out_ref[bb, 0] + n
    out_ref[bb, 1] = out_ref[bb, 1] + s1
    out_ref[bb, 2] = out_ref[bb, 2] + s2


_tc_stats = pl.pallas_call(
    _tc_stats_body,
    out_shape=jax.ShapeDtypeStruct((B, L), jnp.float32),
    grid=(B, NBT),
    in_specs=[
        pl.BlockSpec((RBT, W), lambda b, i: (b * TBLK + S_SC // RBT + i, 0))
    ],
    out_specs=pl.BlockSpec(memory_space=pltpu.SMEM),
)


@functools.partial(
    pl.kernel,
    out_type=jax.ShapeDtypeStruct((R, W), jnp.float32),
    mesh=_mesh,
    scratch_types=[
        pltpu.VMEM((RB2, W), jnp.float32),
        pltpu.VMEM((RB2, W), jnp.float32),
        pltpu.VMEM((RB2, W), jnp.float32),
        pltpu.VMEM((RB2, W), jnp.float32),
        pltpu.VMEM((NW, 3, L), jnp.float32),
        pltpu.VMEM((B, L), jnp.float32),
        pltpu.SemaphoreType.DMA,
        pltpu.SemaphoreType.DMA,
        pltpu.SemaphoreType.DMA,
        pltpu.SemaphoreType.DMA,
    ],
)
def _norm_kernel(
    x_hbm, part_hbm, ptc_hbm, out_hbm,
    in_a, in_b, out_a, out_b, part_v, ptc_v,
    lsem_a, lsem_b, ssem_a, ssem_b,
):
    wid = _worker_id()
    b = wid // WPS
    base = wid * RPW

    pltpu.sync_copy(part_hbm, part_v)
    pltpu.sync_copy(ptc_hbm, ptc_v)
    negv = jnp.zeros((L,), jnp.float32)
    s1v = jnp.zeros((L,), jnp.float32)
    s2v = jnp.zeros((L,), jnp.float32)
    for k in range(WPS):
        w = b * WPS + k
        negv = negv + part_v[w, 0, :]
        s1v = s1v + part_v[w, 1, :]
        s2v = s2v + part_v[w, 2, :]
    neg = negv[0]
    s1 = s1v[0]
    s2 = s2v[0]
    for j in range(1, L):
        neg = neg + negv[j]
        s1 = s1 + s1v[j]
        s2 = s2 + s2v[j]

    tcv = ptc_v[b, pl.ds(0, L)]
    n_sc = float(S_SC * W) - neg
    n_v = jnp.full((L,), n_sc, jnp.float32) + jnp.full((L,), tcv[0], jnp.float32)
    s1_v = jnp.full((L,), s1, jnp.float32) + jnp.full((L,), tcv[1], jnp.float32)
    s2_v = jnp.full((L,), s2, jnp.float32) + jnp.full((L,), tcv[2], jnp.float32)
    mean_v = s1_v / n_v
    var_v = (s2_v - s1_v * mean_v) / (n_v - 1.0)
    var_v = jnp.maximum(var_v, 1e-20)
    bits = lax.bitcast_convert_type(var_v, jnp.int32)
    r = lax.bitcast_convert_type(0x5F3759DF - (bits >> 1), jnp.float32)
    for _ in range(3):
        r = r * (1.5 - 0.5 * var_v * r * r)
    inv_v = 1.0 / (var_v * r + 1e-5)
    c_v = -mean_v * inv_v

    def load(ci, buf, sem):
        return pltpu.make_async_copy(
            x_hbm.at[pl.ds(base + ci * RB2, RB2)], buf, sem
        )

    def store(ci, buf, sem):
        return pltpu.make_async_copy(
            buf, out_hbm.at[pl.ds(base + ci * RB2, RB2)], sem
        )

    def chunk_norm(ibuf, obuf):
        def row_body(j):
            for k in range(KV):
                v = ibuf[j, pl.ds(k * L, L)]
                obuf[j, pl.ds(k * L, L)] = jnp.where(
                    v >= 0.0, v * inv_v + c_v, v
                )

        plsc.parallel_loop(0, RB2, 1)(row_body)

    load(0, in_a, lsem_a).start()

    def pair_body(j, carry):
        load(2 * j + 1, in_b, lsem_b).start()
        load(2 * j, in_a, lsem_a).wait()

        @pl.when(j > 0)
        def _():
            store(2 * j - 2, out_a, ssem_a).wait()

        chunk_norm(in_a, out_a)
        store(2 * j, out_a, ssem_a).start()

        @pl.when(j < NJ2 - 1)
        def _():
            load(2 * j + 2, in_a, lsem_a).start()

        load(2 * j + 1, in_b, lsem_b).wait()

        @pl.when(j > 0)
        def _():
            store(2 * j - 1, out_b, ssem_b).wait()

        chunk_norm(in_b, out_b)
        store(2 * j + 1, out_b, ssem_b).start()
        return carry

    lax.fori_loop(0, NJ2, pair_body, 0)
    store(2 * NJ2 - 2, out_a, ssem_a).wait()
    store(2 * NJ2 - 1, out_b, ssem_b).wait()


def kernel(x):
    x2 = x.reshape(R, W)
    part_tc = _tc_stats(x2)
    part_sc = _stats_kernel(x2)
    out = _norm_kernel(x2, part_sc, part_tc)
    return out.reshape(x.shape)


# speedup vs baseline: 1.4056x; 1.1378x over previous
import functools

import jax
import jax.numpy as jnp
from jax import lax
from jax.experimental import pallas as pl
from jax.experimental.pallas import tpu as pltpu
from jax.experimental.pallas import tpu_sc as plsc

B = 8
C = 96
W = 224
R = B * C * W
E = C * W * W
NC, NS, L = 2, 16, 16
NW = NC * NS
WPS = NW // B
RPW = R // NW
KV = W // L

RB1 = 168
RPS = C * W
S_SC = 13440
RPW1 = S_SC // WPS
NJ1 = RPW1 // RB1 // 2
RBT = 896
NBT = (RPS - S_SC) // RBT
TBLK = RPS // RBT
RB2 = 96
NJ2 = RPW // RB2 // 2

_mesh = plsc.VectorSubcoreMesh(
    core_axis_name="c", subcore_axis_name="s", num_cores=NC, num_subcores=NS
)


def _worker_id():
    return lax.axis_index("s") * NC + lax.axis_index("c")


@functools.partial(
    pl.kernel,
    out_type=jax.ShapeDtypeStruct((NW, 3, L), jnp.float32),
    mesh=_mesh,
    scratch_types=[
        pltpu.VMEM((RB1, W), jnp.float32),
        pltpu.VMEM((RB1, W), jnp.float32),
        pltpu.VMEM((3, L), jnp.float32),
        pltpu.SemaphoreType.DMA,
        pltpu.SemaphoreType.DMA,
    ],
)
def _stats_kernel(x_hbm, part_hbm, buf_a, buf_b, part_v, sem_a, sem_b):
    wid = _worker_id()
    b = wid // WPS
    base = b * RPS + (wid % WPS) * RPW1

    def load(ci, buf, sem):
        return pltpu.make_async_copy(
            x_hbm.at[pl.ds(base + ci * RB1, RB1)], buf, sem
        )

    def chunk_stats(buf, acc):
        def row_body(j, c):
            neg, s1, s2 = c
            for k in range(KV):
                v = buf[j, pl.ds(k * L, L)]
                m = jnp.maximum(v, 0.0)
                neg = neg + lax.shift_right_logical(
                    lax.bitcast_convert_type(v, jnp.int32), 31
                )
                s1 = s1 + m
                s2 = s2 + m * m
            return (neg, s1, s2)

        return plsc.parallel_loop(0, RB1, 1, carry=acc)(row_body)

    load(0, buf_a, sem_a).start()

    def pair_body(j, acc):
        load(2 * j + 1, buf_b, sem_b).start()
        load(2 * j, buf_a, sem_a).wait()
        acc = chunk_stats(buf_a, acc)

        @pl.when(j < NJ1 - 1)
        def _():
            load(2 * j + 2, buf_a, sem_a).start()

        load(2 * j + 1, buf_b, sem_b).wait()
        return chunk_stats(buf_b, acc)

    zf = jnp.zeros((L,), jnp.float32)
    zi = jnp.zeros((L,), jnp.int32)
    neg, s1, s2 = lax.fori_loop(0, NJ1, pair_body, (zi, zf, zf))
    part_v[0, :] = neg.astype(jnp.float32)
    part_v[1, :] = s1
    part_v[2, :] = s2
    pltpu.sync_copy(part_v, part_hbm.at[wid])


def _tc_stats_body(x_ref, out_ref):
    bb = pl.program_id(0)
    i = pl.program_id(1)
    v = x_ref[...]
    valid = (v >= 0.0).astype(jnp.float32)
    m = jnp.maximum(v, 0.0)
    n = jnp.sum(valid)
    s1 = jnp.sum(m)
    s2 = jnp.sum(m * m)

    @pl.when(i == 0)
    def _():
        for j in range(L):
            out_ref[bb, j] = 0.0

    out_ref[bb, 0] = out_ref[bb, 0] + n
    out_ref[bb, 1] = out_ref[bb, 1] + s1
    out_ref[bb, 2] = out_ref[bb, 2] + s2


_tc_stats = pl.pallas_call(
    _tc_stats_body,
    out_shape=jax.ShapeDtypeStruct((B, L), jnp.float32),
    grid=(B, NBT),
    in_specs=[
        pl.BlockSpec((RBT, W), lambda b, i: (b * TBLK + S_SC // RBT + i, 0))
    ],
    out_specs=pl.BlockSpec(memory_space=pltpu.SMEM),
)


@functools.partial(
    pl.kernel,
    out_type=jax.ShapeDtypeStruct((R, W), jnp.float32),
    mesh=_mesh,
    scratch_types=[
        pltpu.VMEM((RB2, W), jnp.float32),
        pltpu.VMEM((RB2, W), jnp.float32),
        pltpu.VMEM((RB2, W), jnp.float32),
        pltpu.VMEM((RB2, W), jnp.float32),
        pltpu.VMEM((NW, 3, L), jnp.float32),
        pltpu.VMEM((B, L), jnp.float32),
        pltpu.SemaphoreType.DMA,
        pltpu.SemaphoreType.DMA,
        pltpu.SemaphoreType.DMA,
        pltpu.SemaphoreType.DMA,
    ],
)
def _norm_kernel(
    x_hbm, part_hbm, ptc_hbm, out_hbm,
    in_a, in_b, out_a, out_b, part_v, ptc_v,
    lsem_a, lsem_b, ssem_a, ssem_b,
):
    wid = _worker_id()
    b = wid // WPS
    base = wid * RPW

    pltpu.sync_copy(part_hbm, part_v)
    pltpu.sync_copy(ptc_hbm, ptc_v)
    negv = jnp.zeros((L,), jnp.float32)
    s1v = jnp.zeros((L,), jnp.float32)
    s2v = jnp.zeros((L,), jnp.float32)
    for k in range(WPS):
        w = b * WPS + k
        negv = negv + part_v[w, 0, :]
        s1v = s1v + part_v[w, 1, :]
        s2v = s2v + part_v[w, 2, :]
    neg = negv[0]
    s1 = s1v[0]
    s2 = s2v[0]
    for j in range(1, L):
        neg = neg + negv[j]
        s1 = s1 + s1v[j]
        s2 = s2 + s2v[j]

    tcv = ptc_v[b, pl.ds(0, L)]
    n_sc = float(S_SC * W) - neg
    n_v = jnp.full((L,), n_sc, jnp.float32) + jnp.full((L,), tcv[0], jnp.float32)
    s1_v = jnp.full((L,), s1, jnp.float32) + jnp.full((L,), tcv[1], jnp.float32)
    s2_v = jnp.full((L,), s2, jnp.float32) + jnp.full((L,), tcv[2], jnp.float32)
    mean_v = s1_v / n_v
    var_v = (s2_v - s1_v * mean_v) / (n_v - 1.0)
    var_v = jnp.maximum(var_v, 1e-20)
    bits = lax.bitcast_convert_type(var_v, jnp.int32)
    r = lax.bitcast_convert_type(0x5F3759DF - (bits >> 1), jnp.float32)
    for _ in range(3):
        r = r * (1.5 - 0.5 * var_v * r * r)
    inv_v = 1.0 / (var_v * r + 1e-5)
    c_v = -mean_v * inv_v

    def load(ci, buf, sem):
        return pltpu.make_async_copy(
            x_hbm.at[pl.ds(base + ci * RB2, RB2)], buf, sem
        )

    def store(ci, buf, sem):
        return pltpu.make_async_copy(
            buf, out_hbm.at[pl.ds(base + ci * RB2, RB2)], sem
        )

    def chunk_norm(ibuf, obuf):
        def row_body(j):
            for k in range(KV):
                v = ibuf[j, pl.ds(k * L, L)]
                obuf[j, pl.ds(k * L, L)] = jnp.where(
                    v >= 0.0, v * inv_v + c_v, v
                )

        plsc.parallel_loop(0, RB2, 1)(row_body)

    load(0, in_a, lsem_a).start()

    def pair_body(j, carry):
        load(2 * j + 1, in_b, lsem_b).start()
        load(2 * j, in_a, lsem_a).wait()

        @pl.when(j > 0)
        def _():
            store(2 * j - 2, out_a, ssem_a).wait()

        chunk_norm(in_a, out_a)
        store(2 * j, out_a, ssem_a).start()

        @pl.when(j < NJ2 - 1)
        def _():
            load(2 * j + 2, in_a, lsem_a).start()

        load(2 * j + 1, in_b, lsem_b).wait()

        @pl.when(j > 0)
        def _():
            store(2 * j - 1, out_b, ssem_b).wait()

        chunk_norm(in_b, out_b)
        store(2 * j + 1, out_b, ssem_b).start()
        return carry

    lax.fori_loop(0, NJ2, pair_body, 0)
    store(2 * NJ2 - 2, out_a, ssem_a).wait()
    store(2 * NJ2 - 1, out_b, ssem_b).wait()


def kernel(x):
    x2 = x.reshape(R, W)
    part_tc = _tc_stats(x2)
    part_sc = _stats_kernel(x2)
    out = _norm_kernel(x2, part_sc, part_tc)
    return out.reshape(x.shape)


# speedup vs baseline: 1.4197x; 1.0100x over previous
import functools

import jax
import jax.numpy as jnp
from jax import lax
from jax.experimental import pallas as pl
from jax.experimental.pallas import tpu as pltpu
from jax.experimental.pallas import tpu_sc as plsc

B = 8
C = 96
W = 224
R = B * C * W
E = C * W * W
NC, NS, L = 2, 16, 16
NW = NC * NS
WPS = NW // B
RPW = R // NW
KV = W // L

RB1 = 224
RPS = C * W
S_SC = 14336
RPW1 = S_SC // WPS
NJ1 = RPW1 // RB1 // 2
RBT = 896
NBT = (RPS - S_SC) // RBT
TBLK = RPS // RBT
RB2 = 96
NJ2 = RPW // RB2 // 2

_mesh = plsc.VectorSubcoreMesh(
    core_axis_name="c", subcore_axis_name="s", num_cores=NC, num_subcores=NS
)


def _worker_id():
    return lax.axis_index("s") * NC + lax.axis_index("c")


@functools.partial(
    pl.kernel,
    out_type=jax.ShapeDtypeStruct((NW, 3, L), jnp.float32),
    mesh=_mesh,
    scratch_types=[
        pltpu.VMEM((RB1, W), jnp.float32),
        pltpu.VMEM((RB1, W), jnp.float32),
        pltpu.VMEM((3, L), jnp.float32),
        pltpu.SemaphoreType.DMA,
        pltpu.SemaphoreType.DMA,
    ],
)
def _stats_kernel(x_hbm, part_hbm, buf_a, buf_b, part_v, sem_a, sem_b):
    wid = _worker_id()
    b = wid // WPS
    base = b * RPS + (wid % WPS) * RPW1

    def load(ci, buf, sem):
        return pltpu.make_async_copy(
            x_hbm.at[pl.ds(base + ci * RB1, RB1)], buf, sem
        )

    def chunk_stats(buf, acc):
        def row_body(j, c):
            neg, s1, s2 = c
            for k in range(KV):
                v = buf[j, pl.ds(k * L, L)]
                m = jnp.maximum(v, 0.0)
                neg = neg + lax.shift_right_logical(
                    lax.bitcast_convert_type(v, jnp.int32), 31
                )
                s1 = s1 + m
                s2 = s2 + m * m
            return (neg, s1, s2)

        return plsc.parallel_loop(0, RB1, 1, carry=acc)(row_body)

    load(0, buf_a, sem_a).start()

    def pair_body(j, acc):
        load(2 * j + 1, buf_b, sem_b).start()
        load(2 * j, buf_a, sem_a).wait()
        acc = chunk_stats(buf_a, acc)

        @pl.when(j < NJ1 - 1)
        def _():
            load(2 * j + 2, buf_a, sem_a).start()

        load(2 * j + 1, buf_b, sem_b).wait()
        return chunk_stats(buf_b, acc)

    zf = jnp.zeros((L,), jnp.float32)
    zi = jnp.zeros((L,), jnp.int32)
    neg, s1, s2 = lax.fori_loop(0, NJ1, pair_body, (zi, zf, zf))
    part_v[0, :] = neg.astype(jnp.float32)
    part_v[1, :] = s1
    part_v[2, :] = s2
    pltpu.sync_copy(part_v, part_hbm.at[wid])


def _tc_stats_body(x_ref, out_ref):
    bb = pl.program_id(0)
    i = pl.program_id(1)
    v = x_ref[...]
    valid = (v >= 0.0).astype(jnp.float32)
    m = jnp.maximum(v, 0.0)
    n = jnp.sum(valid)
    s1 = jnp.sum(m)
    s2 = jnp.sum(m * m)

    @pl.when(i == 0)
    def _():
        for j in range(L):
            out_ref[bb, j] = 0.0

    out_ref[bb, 0] = out_ref[bb, 0] + n
    out_ref[bb, 1] = out_ref[bb, 1] + s1
    out_ref[bb, 2] = out_ref[bb, 2] + s2


_tc_stats = pl.pallas_call(
    _tc_stats_body,
    out_shape=jax.ShapeDtypeStruct((B, L), jnp.float32),
    grid=(B, NBT),
    in_specs=[
        pl.BlockSpec((RBT, W), lambda b, i: (b * TBLK + S_SC // RBT + i, 0))
    ],
    out_specs=pl.BlockSpec(memory_space=pltpu.SMEM),
)


@functools.partial(
    pl.kernel,
    out_type=jax.ShapeDtypeStruct((R, W), jnp.float32),
    mesh=_mesh,
    scratch_types=[
        pltpu.VMEM((RB2, W), jnp.float32),
        pltpu.VMEM((RB2, W), jnp.float32),
        pltpu.VMEM((RB2, W), jnp.float32),
        pltpu.VMEM((RB2, W), jnp.float32),
        pltpu.VMEM((NW, 3, L), jnp.float32),
        pltpu.VMEM((B, L), jnp.float32),
        pltpu.SemaphoreType.DMA,
        pltpu.SemaphoreType.DMA,
        pltpu.SemaphoreType.DMA,
        pltpu.SemaphoreType.DMA,
    ],
)
def _norm_kernel(
    x_hbm, part_hbm, ptc_hbm, out_hbm,
    in_a, in_b, out_a, out_b, part_v, ptc_v,
    lsem_a, lsem_b, ssem_a, ssem_b,
):
    wid = _worker_id()
    b = wid // WPS
    base = wid * RPW

    pltpu.sync_copy(part_hbm, part_v)
    pltpu.sync_copy(ptc_hbm, ptc_v)
    negv = jnp.zeros((L,), jnp.float32)
    s1v = jnp.zeros((L,), jnp.float32)
    s2v = jnp.zeros((L,), jnp.float32)
    for k in range(WPS):
        w = b * WPS + k
        negv = negv + part_v[w, 0, :]
        s1v = s1v + part_v[w, 1, :]
        s2v = s2v + part_v[w, 2, :]
    neg = negv[0]
    s1 = s1v[0]
    s2 = s2v[0]
    for j in range(1, L):
        neg = neg + negv[j]
        s1 = s1 + s1v[j]
        s2 = s2 + s2v[j]

    tcv = ptc_v[b, pl.ds(0, L)]
    n_sc = float(S_SC * W) - neg
    n_v = jnp.full((L,), n_sc, jnp.float32) + jnp.full((L,), tcv[0], jnp.float32)
    s1_v = jnp.full((L,), s1, jnp.float32) + jnp.full((L,), tcv[1], jnp.float32)
    s2_v = jnp.full((L,), s2, jnp.float32) + jnp.full((L,), tcv[2], jnp.float32)
    mean_v = s1_v / n_v
    var_v = (s2_v - s1_v * mean_v) / (n_v - 1.0)
    var_v = jnp.maximum(var_v, 1e-20)
    bits = lax.bitcast_convert_type(var_v, jnp.int32)
    r = lax.bitcast_convert_type(0x5F3759DF - (bits >> 1), jnp.float32)
    for _ in range(3):
        r = r * (1.5 - 0.5 * var_v * r * r)
    inv_v = 1.0 / (var_v * r + 1e-5)
    c_v = -mean_v * inv_v

    def load(ci, buf, sem):
        return pltpu.make_async_copy(
            x_hbm.at[pl.ds(base + ci * RB2, RB2)], buf, sem
        )

    def store(ci, buf, sem):
        return pltpu.make_async_copy(
            buf, out_hbm.at[pl.ds(base + ci * RB2, RB2)], sem
        )

    def chunk_norm(ibuf, obuf):
        def row_body(j):
            for k in range(KV):
                v = ibuf[j, pl.ds(k * L, L)]
                obuf[j, pl.ds(k * L, L)] = jnp.where(
                    v >= 0.0, v * inv_v + c_v, v
                )

        plsc.parallel_loop(0, RB2, 1)(row_body)

    load(0, in_a, lsem_a).start()

    def pair_body(j, carry):
        load(2 * j + 1, in_b, lsem_b).start()
        load(2 * j, in_a, lsem_a).wait()

        @pl.when(j > 0)
        def _():
            store(2 * j - 2, out_a, ssem_a).wait()

        chunk_norm(in_a, out_a)
        store(2 * j, out_a, ssem_a).start()

        @pl.when(j < NJ2 - 1)
        def _():
            load(2 * j + 2, in_a, lsem_a).start()

        load(2 * j + 1, in_b, lsem_b).wait()

        @pl.when(j > 0)
        def _():
            store(2 * j - 1, out_b, ssem_b).wait()

        chunk_norm(in_b, out_b)
        store(2 * j + 1, out_b, ssem_b).start()
        return carry

    lax.fori_loop(0, NJ2, pair_body, 0)
    store(2 * NJ2 - 2, out_a, ssem_a).wait()
    store(2 * NJ2 - 1, out_b, ssem_b).wait()


def kernel(x):
    x2 = x.reshape(R, W)
    part_tc = _tc_stats(x2)
    part_sc = _stats_kernel(x2)
    out = _norm_kernel(x2, part_sc, part_tc)
    return out.reshape(x.shape)
